# baseline (speedup 1.0000x reference)
import jax
import jax.numpy as jnp
from jax.experimental import pallas as pl
from jax.experimental.pallas import tpu as pltpu


def _bf(x):
    return x.astype(jnp.bfloat16)


def _attn_kernel(x_full_ref, x_q_ref, wt_ref, bt_ref, wp_ref, bp_ref,
                 wg_ref, bg_ref, wo_ref, bo_ref, out_ref, phi_ref, g_ref):
    q = pl.program_id(1)

    @pl.when(q == 0)
    def _proj():
        xb = _bf(x_full_ref[0])
        phi_ref[...] = _bf(jnp.dot(_bf(wp_ref[...]), xb,
                                   preferred_element_type=jnp.float32)
                           + bp_ref[...])
        g_ref[...] = _bf(jnp.dot(_bf(wg_ref[...]), xb,
                                 preferred_element_type=jnp.float32)
                         + bg_ref[...])

    xq = x_q_ref[0]
    theta = _bf(jnp.dot(_bf(wt_ref[...]), _bf(xq),
                        preferred_element_type=jnp.float32)
                + bt_ref[...])
    f = jax.lax.dot_general(theta, phi_ref[...], (((0,), (0,)), ((), ())),
                            preferred_element_type=jnp.float32) * 10.0
    m = jnp.max(f, axis=1, keepdims=True)
    e = jnp.exp(f - m)
    s = jnp.sum(e, axis=1, keepdims=True)
    attn = _bf(e / s)
    y = _bf(jax.lax.dot_general(attn, g_ref[...], (((1,), (1,)), ((), ())),
                                preferred_element_type=jnp.float32))
    o = jax.lax.dot_general(_bf(wo_ref[...]), y, (((1,), (1,)), ((), ())),
                            preferred_element_type=jnp.float32)
    out_ref[0] = o + bo_ref[...] + xq


def kernel(vid, W_theta, b_theta, W_phi, b_phi, W_g, b_g, W_out, b_out):
    B, C, H, Wd = vid.shape
    N = H * Wd
    inter = W_theta.shape[0]
    QBLK = 512
    x = vid.reshape(B, C, N)
    bt = b_theta.reshape(inter, 1)
    bp = b_phi.reshape(inter, 1)
    bg = b_g.reshape(inter, 1)
    bo = b_out.reshape(C, 1)
    out = pl.pallas_call(
        _attn_kernel,
        grid=(B, N // QBLK),
        in_specs=[
            pl.BlockSpec((1, C, N), lambda b, q: (b, 0, 0)),
            pl.BlockSpec((1, C, QBLK), lambda b, q: (b, 0, q)),
            pl.BlockSpec((inter, C), lambda b, q: (0, 0)),
            pl.BlockSpec((inter, 1), lambda b, q: (0, 0)),
            pl.BlockSpec((inter, C), lambda b, q: (0, 0)),
            pl.BlockSpec((inter, 1), lambda b, q: (0, 0)),
            pl.BlockSpec((inter, C), lambda b, q: (0, 0)),
            pl.BlockSpec((inter, 1), lambda b, q: (0, 0)),
            pl.BlockSpec((C, inter), lambda b, q: (0, 0)),
            pl.BlockSpec((C, 1), lambda b, q: (0, 0)),
        ],
        out_specs=pl.BlockSpec((1, C, QBLK), lambda b, q: (b, 0, q)),
        out_shape=jax.ShapeDtypeStruct((B, C, N), jnp.float32),
        scratch_shapes=[
            pltpu.VMEM((inter, N), jnp.bfloat16),
            pltpu.VMEM((inter, N), jnp.bfloat16),
        ],
    )(x, x, W_theta, bt, W_phi, bp, W_g, bg, W_out, bo)
    return out.reshape(B, C, H, Wd)


# speedup vs baseline: 1.7240x; 1.7240x over previous
import jax
import jax.numpy as jnp
from jax.experimental import pallas as pl
from jax.experimental.pallas import tpu as pltpu


def _bf(x):
    return x.astype(jnp.bfloat16)


def _attn_kernel(x_full_ref, x_q_ref, wt_ref, bt_ref, wp_ref, bp_ref,
                 wg_ref, bg_ref, wo_ref, bo_ref, out_ref, phi_ref, go_ref):
    C = x_q_ref.shape[1]
    q = pl.program_id(1)

    @pl.when(q == 0)
    def _proj():
        xb = _bf(x_full_ref[0])
        phi_ref[...] = _bf(jnp.dot(_bf(wp_ref[...]), xb,
                                   preferred_element_type=jnp.float32)
                           + bp_ref[...])
        g = _bf(jnp.dot(_bf(wg_ref[...]), xb,
                        preferred_element_type=jnp.float32) + bg_ref[...])
        go_ref[:C, :] = _bf(jnp.dot(_bf(wo_ref[...]), g,
                                    preferred_element_type=jnp.float32))
        go_ref[C:, :] = jnp.ones_like(go_ref[C:, :])

    xq = x_q_ref[0]
    theta = _bf(jnp.dot(_bf(wt_ref[...]), _bf(xq),
                        preferred_element_type=jnp.float32)
                + bt_ref[...])
    f = jax.lax.dot_general(theta, phi_ref[...], (((0,), (0,)), ((), ())),
                            preferred_element_type=jnp.float32) * 10.0
    m = jnp.max(f, axis=1, keepdims=True)
    e = _bf(jnp.exp(f - m))
    o = jax.lax.dot_general(go_ref[...], e, (((1,), (1,)), ((), ())),
                            preferred_element_type=jnp.float32)
    out_ref[0] = o[:C, :] / o[C:C + 1, :] + bo_ref[...] + xq


def kernel(vid, W_theta, b_theta, W_phi, b_phi, W_g, b_g, W_out, b_out):
    B, C, H, Wd = vid.shape
    N = H * Wd
    inter = W_theta.shape[0]
    QBLK = 512
    x = vid.reshape(B, C, N)
    bt = b_theta.reshape(inter, 1)
    bp = b_phi.reshape(inter, 1)
    bg = b_g.reshape(inter, 1)
    bo = b_out.reshape(C, 1)
    out = pl.pallas_call(
        _attn_kernel,
        grid=(B, N // QBLK),
        in_specs=[
            pl.BlockSpec((1, C, N), lambda b, q: (b, 0, 0)),
            pl.BlockSpec((1, C, QBLK), lambda b, q: (b, 0, q)),
            pl.BlockSpec((inter, C), lambda b, q: (0, 0)),
            pl.BlockSpec((inter, 1), lambda b, q: (0, 0)),
            pl.BlockSpec((inter, C), lambda b, q: (0, 0)),
            pl.BlockSpec((inter, 1), lambda b, q: (0, 0)),
            pl.BlockSpec((inter, C), lambda b, q: (0, 0)),
            pl.BlockSpec((inter, 1), lambda b, q: (0, 0)),
            pl.BlockSpec((C, inter), lambda b, q: (0, 0)),
            pl.BlockSpec((C, 1), lambda b, q: (0, 0)),
        ],
        out_specs=pl.BlockSpec((1, C, QBLK), lambda b, q: (b, 0, q)),
        out_shape=jax.ShapeDtypeStruct((B, C, N), jnp.float32),
        scratch_shapes=[
            pltpu.VMEM((inter, N), jnp.bfloat16),
            pltpu.VMEM((C + 8, N), jnp.bfloat16),
        ],
    )(x, x, W_theta, bt, W_phi, bp, W_g, bg, W_out, bo)
    return out.reshape(B, C, H, Wd)


# speedup vs baseline: 2.0557x; 1.1924x over previous
import jax
import jax.numpy as jnp
from jax.experimental import pallas as pl
from jax.experimental.pallas import tpu as pltpu


def _bf(x):
    return x.astype(jnp.bfloat16)


def _attn_kernel(x_full_ref, x_q_ref, wt_ref, bt_ref, wp_ref, bp_ref,
                 wg_ref, bg_ref, wo_ref, bo_ref, out_ref, phi_ref, go_ref):
    C = x_q_ref.shape[1]
    q = pl.program_id(1)

    @pl.when(q == 0)
    def _proj():
        xb = _bf(x_full_ref[0])
        phi_ref[...] = _bf(jnp.dot(_bf(wp_ref[...]), xb,
                                   preferred_element_type=jnp.float32)
                           + bp_ref[...])
        g = _bf(jnp.dot(_bf(wg_ref[...]), xb,
                        preferred_element_type=jnp.float32) + bg_ref[...])
        go_ref[:C, :] = _bf(jnp.dot(_bf(wo_ref[...]), g,
                                    preferred_element_type=jnp.float32))
        go_ref[C:, :] = jnp.ones_like(go_ref[C:, :])

    xq = x_q_ref[0]
    theta = _bf(jnp.dot(_bf(wt_ref[...]), _bf(xq),
                        preferred_element_type=jnp.float32)
                + bt_ref[...])
    f = jax.lax.dot_general(theta, phi_ref[...], (((0,), (0,)), ((), ())),
                            preferred_element_type=jnp.float32)
    m = jnp.max(f, axis=1, keepdims=True)
    e = _bf(jnp.exp2((f - m) * 14.426950408889634))
    o = jax.lax.dot_general(go_ref[...], e, (((1,), (1,)), ((), ())),
                            preferred_element_type=jnp.float32)
    out_ref[0] = o[:C, :] / o[C:C + 1, :] + bo_ref[...] + xq


def kernel(vid, W_theta, b_theta, W_phi, b_phi, W_g, b_g, W_out, b_out):
    B, C, H, Wd = vid.shape
    N = H * Wd
    inter = W_theta.shape[0]
    QBLK = 1024
    x = vid.reshape(B, C, N)
    bt = b_theta.reshape(inter, 1)
    bp = b_phi.reshape(inter, 1)
    bg = b_g.reshape(inter, 1)
    bo = b_out.reshape(C, 1)
    out = pl.pallas_call(
        _attn_kernel,
        grid=(B, N // QBLK),
        in_specs=[
            pl.BlockSpec((1, C, N), lambda b, q: (b, 0, 0)),
            pl.BlockSpec((1, C, QBLK), lambda b, q: (b, 0, q)),
            pl.BlockSpec((inter, C), lambda b, q: (0, 0)),
            pl.BlockSpec((inter, 1), lambda b, q: (0, 0)),
            pl.BlockSpec((inter, C), lambda b, q: (0, 0)),
            pl.BlockSpec((inter, 1), lambda b, q: (0, 0)),
            pl.BlockSpec((inter, C), lambda b, q: (0, 0)),
            pl.BlockSpec((inter, 1), lambda b, q: (0, 0)),
            pl.BlockSpec((C, inter), lambda b, q: (0, 0)),
            pl.BlockSpec((C, 1), lambda b, q: (0, 0)),
        ],
        out_specs=pl.BlockSpec((1, C, QBLK), lambda b, q: (b, 0, q)),
        out_shape=jax.ShapeDtypeStruct((B, C, N), jnp.float32),
        scratch_shapes=[
            pltpu.VMEM((inter, N), jnp.bfloat16),
            pltpu.VMEM((C + 8, N), jnp.bfloat16),
        ],
    )(x, x, W_theta, bt, W_phi, bp, W_g, bg, W_out, bo)
    return out.reshape(B, C, H, Wd)
